# baseline (speedup 1.0000x reference)
import functools

import jax
import jax.numpy as jnp
from jax import lax
from jax.experimental import pallas as pl
from jax.experimental.pallas import tpu as pltpu
from jax.experimental.pallas import tpu_sc as plsc

N = 50000
NPAD = 50176
E = 800000
EPAD = 802816
D = 64
R = 3
G = 128
CHALF = NPAD // 2
DUMP = CHALF
CDUMP = 3 * CHALF
GP = 136
PDUMP = G
BN = 512
NB = 98
RPT = NPAD // 32
EPT = EPAD // 16
SUP = 512
NSUP = EPT // SUP
CHUNK = 128
F32 = jnp.float32
I32 = jnp.int32

_mesh = plsc.VectorSubcoreMesh(core_axis_name="c", subcore_axis_name="s")
_SC_PARAMS = pltpu.CompilerParams(use_tc_tiling_on_sc=False,
                                  needs_layout_passes=False)


def _zero16():
    return jnp.zeros((16,), F32)


def _fill_zero_rows(ref, nrows):
    def body(j, carry):
        for c4 in range(4):
            ref[j, pl.ds(16 * c4, 16)] = _zero16()
        return carry
    lax.fori_loop(0, nrows, body, 0)


def _fill_zero_flat(ref, n):
    def body(k, carry):
        ref[pl.ds(k * 16, 16)] = _zero16()
        return carry
    lax.fori_loop(0, n // 16, body, 0)


def _embed_body(x_hbm, tab_hbm, h_hbm, x_v, rows_v, sem):
    cid = lax.axis_index("c")
    sid = lax.axis_index("s")
    wid = cid * 16 + sid
    base = wid * RPT
    pltpu.sync_copy(x_hbm.at[pl.ds(base, RPT)], x_v)

    def chunk(c, carry):
        off = c * 112
        pltpu.async_copy(tab_hbm.at[x_v.at[pl.ds(off, 112)]], rows_v, sem).wait()
        pltpu.sync_copy(rows_v, h_hbm.at[pl.ds(base + off, 112)])
        return carry
    lax.fori_loop(0, RPT // 112, chunk, 0)


def _embed_call(x_pad, embed_table):
    return pl.kernel(
        _embed_body,
        out_type=jax.ShapeDtypeStruct((NPAD, D), F32),
        mesh=_mesh,
        compiler_params=_SC_PARAMS,
        scratch_types=[
            pltpu.VMEM((RPT,), I32),
            pltpu.VMEM((112, D), F32),
            pltpu.SemaphoreType.DMA,
        ],
    )(x_pad, embed_table)


def _mm_body(h_ref, w_ref, b_ref, hroot_ref, t_ref):
    prod = lax.dot_general(h_ref[...], w_ref[...], (((1,), (0,)), ((), ())),
                           preferred_element_type=F32)
    hroot_ref[...] = prod[:, 0:64] + b_ref[...]
    t_ref[0] = prod[:, 64:128]
    t_ref[1] = prod[:, 128:192]
    t_ref[2] = prod[:, 192:256]


def _mm_call(h, wcat, b):
    return pl.pallas_call(
        _mm_body,
        grid=(NB,),
        in_specs=[
            pl.BlockSpec((BN, D), lambda i: (i, 0)),
            pl.BlockSpec((D, 4 * D), lambda i: (0, 0)),
            pl.BlockSpec((1, D), lambda i: (0, 0)),
        ],
        out_specs=[
            pl.BlockSpec((BN, D), lambda i: (i, 0)),
            pl.BlockSpec((3, BN, D), lambda i: (0, i, 0)),
        ],
        out_shape=[
            jax.ShapeDtypeStruct((NPAD, D), F32),
            jax.ShapeDtypeStruct((3, NPAD, D), F32),
        ],
    )(h, wcat, b)


def _nmm_body(hroot_ref, a_ref, w_ref, b_ref, hroot2_ref, t_ref):
    hb = jnp.maximum(hroot_ref[...] + a_ref[...], 0.0)
    prod = lax.dot_general(hb, w_ref[...], (((1,), (0,)), ((), ())),
                           preferred_element_type=F32)
    hroot2_ref[...] = prod[:, 0:64] + b_ref[...]
    t_ref[0] = prod[:, 64:128]
    t_ref[1] = prod[:, 128:192]
    t_ref[2] = prod[:, 192:256]


def _nmm_call(hroot, acc, wcat, b):
    return pl.pallas_call(
        _nmm_body,
        grid=(NB,),
        in_specs=[
            pl.BlockSpec((BN, D), lambda i: (i, 0)),
            pl.BlockSpec((BN, D), lambda i: (i, 0)),
            pl.BlockSpec((D, 4 * D), lambda i: (0, 0)),
            pl.BlockSpec((1, D), lambda i: (0, 0)),
        ],
        out_specs=[
            pl.BlockSpec((BN, D), lambda i: (i, 0)),
            pl.BlockSpec((3, BN, D), lambda i: (0, i, 0)),
        ],
        out_shape=[
            jax.ShapeDtypeStruct((NPAD, D), F32),
            jax.ShapeDtypeStruct((3, NPAD, D), F32),
        ],
    )(hroot, acc, wcat, b)


def _cnt_body(src_hbm, dst_hbm, typ_hbm, inv_hbm,
              src_v, dst_v, typ_v, cq_v, ones_v, invrow_v, cnt_sh):
    cid = lax.axis_index("c")
    sid = lax.axis_index("s")

    for k in range(CHUNK // 16):
        ones_v[pl.ds(16 * k, 16)] = jnp.ones((16,), F32)
    _fill_zero_flat(invrow_v, CHUNK)

    base = sid * (3 * CHALF // 16)

    def zcnt(k, carry):
        pltpu.sync_copy(invrow_v.at[pl.ds(0, 112)],
                        cnt_sh.at[pl.ds(base + k * 112, 112)])
        return carry
    lax.fori_loop(0, 3 * CHALF // 16 // 112, zcnt, 0)

    @pl.when(sid == 0)
    def _():
        pltpu.sync_copy(invrow_v.at[pl.ds(0, 16)],
                        cnt_sh.at[pl.ds(CDUMP, 16)])

    plsc.subcore_barrier()

    def sup(s, carry):
        ebase = sid * EPT + s * SUP
        pltpu.sync_copy(dst_hbm.at[pl.ds(ebase, SUP)], dst_v)
        pltpu.sync_copy(typ_hbm.at[pl.ds(ebase, SUP)], typ_v)
        for j in range(SUP // CHUNK):
            for i in range(CHUNK // 16):
                o = j * CHUNK + i * 16
                d16 = dst_v[pl.ds(o, 16)]
                t16 = typ_v[pl.ds(o, 16)]
                ld = d16 - cid * CHALF
                inb = (ld >= 0) & (ld < CHALF)
                cq_v[pl.ds(i * 16, 16)] = jnp.where(inb, t16 * CHALF + ld, CDUMP)
            pltpu.sync_copy(ones_v, cnt_sh.at[cq_v], add=True)
        return carry
    lax.fori_loop(0, NSUP, sup, 0)

    plsc.subcore_barrier()

    for r in range(3):
        lb = r * CHALF + sid * (CHALF // 16)
        gb = r * NPAD + cid * CHALF + sid * (CHALF // 16)

        def invc(k, carry, lb=lb, gb=gb):
            pltpu.sync_copy(cnt_sh.at[pl.ds(lb + k * 112, 112)],
                            invrow_v.at[pl.ds(0, 112)])
            for i in range(7):
                c16 = invrow_v[pl.ds(i * 16, 16)]
                invrow_v[pl.ds(i * 16, 16)] = 1.0 / jnp.maximum(c16, 1.0)
            pltpu.sync_copy(invrow_v.at[pl.ds(0, 112)],
                            inv_hbm.at[pl.ds(gb + k * 112, 112)])
            return carry
        lax.fori_loop(0, CHALF // 16 // 112, invc, 0)


def _cnt_call(srcp, dstp, typp):
    return pl.kernel(
        _cnt_body,
        out_type=jax.ShapeDtypeStruct((3 * NPAD,), F32),
        mesh=_mesh,
        compiler_params=_SC_PARAMS,
        scratch_types=[
            pltpu.VMEM((SUP,), I32),
            pltpu.VMEM((SUP,), I32),
            pltpu.VMEM((SUP,), I32),
            pltpu.VMEM((CHUNK,), I32),
            pltpu.VMEM((CHUNK,), F32),
            pltpu.VMEM((CHUNK,), F32),
            pltpu.VMEM_SHARED((3 * CHALF + 16,), F32),
        ],
    )(srcp, dstp, typp)


def _edge_body(t_hbm, src_hbm, dst_hbm, typ_hbm, inv_hbm, acc_hbm,
               src_v, dst_v, typ_v, gidx_a, lidx_a, iidx_a, gidx_b, lidx_b,
               iidx_b, rows_a, rows_b, invr_a, invr_b, zblk_v,
               sem_ra, sem_rb, sem_ia, sem_ib, acc_sh):
    cid = lax.axis_index("c")
    sid = lax.axis_index("s")
    rbase = sid * (CHALF // 16)

    _fill_zero_rows(zblk_v, 16)

    def stage(s):
        ebase = sid * EPT + s * SUP
        pltpu.sync_copy(src_hbm.at[pl.ds(ebase, SUP)], src_v)
        pltpu.sync_copy(dst_hbm.at[pl.ds(ebase, SUP)], dst_v)
        pltpu.sync_copy(typ_hbm.at[pl.ds(ebase, SUP)], typ_v)

    def cidx(off, gidx, lidx, iidx):
        for i in range(CHUNK // 16):
            o = off + i * 16
            s16 = src_v[pl.ds(o, 16)]
            d16 = dst_v[pl.ds(o, 16)]
            t16 = typ_v[pl.ds(o, 16)]
            gidx[pl.ds(i * 16, 16)] = t16 * NPAD + s16
            ld = d16 - cid * CHALF
            inb = (ld >= 0) & (ld < CHALF)
            lidx[pl.ds(i * 16, 16)] = jnp.where(inb, ld, DUMP)
            iidx[pl.ds(i * 16, 16)] = jnp.where(inb, t16 * NPAD + d16, 0)

    def issue(gidx, rows, sem_r, iidx, invr, sem_i):
        pltpu.async_copy(t_hbm.at[gidx], rows, sem_r)
        pltpu.async_copy(inv_hbm.at[iidx], invr, sem_i)

    def drain(gidx, rows, sem_r, iidx, invr, sem_i, lidx):
        pltpu.make_async_copy(t_hbm.at[gidx], rows, sem_r).wait()
        pltpu.make_async_copy(inv_hbm.at[iidx], invr, sem_i).wait()

        def rowb(jr, carry):
            ib = plsc.load_gather(invr, [jnp.zeros((16,), I32) + jr])
            for c4 in range(4):
                sl = pl.ds(16 * c4, 16)
                rows[jr, sl] = rows[jr, sl] * ib
            return carry
        lax.fori_loop(0, CHUNK, rowb, 0)
        pltpu.sync_copy(rows, acc_sh.at[lidx], add=True)

    def zc(c, carry):
        pltpu.sync_copy(zblk_v, acc_sh.at[pl.ds(rbase + c * 16, 16)])
        return carry
    lax.fori_loop(0, CHALF // 16 // 16, zc, 0)

    @pl.when(sid == 0)
    def _():
        pltpu.sync_copy(zblk_v.at[pl.ds(0, 8)], acc_sh.at[pl.ds(CHALF, 8)])

    plsc.subcore_barrier()

    stage(0)
    cidx(0, gidx_a, lidx_a, iidx_a)
    issue(gidx_a, rows_a, sem_ra, iidx_a, invr_a, sem_ia)

    def sup(s, carry):
        cidx(CHUNK, gidx_b, lidx_b, iidx_b)
        issue(gidx_b, rows_b, sem_rb, iidx_b, invr_b, sem_ib)
        drain(gidx_a, rows_a, sem_ra, iidx_a, invr_a, sem_ia, lidx_a)

        cidx(2 * CHUNK, gidx_a, lidx_a, iidx_a)
        issue(gidx_a, rows_a, sem_ra, iidx_a, invr_a, sem_ia)
        drain(gidx_b, rows_b, sem_rb, iidx_b, invr_b, sem_ib, lidx_b)

        cidx(3 * CHUNK, gidx_b, lidx_b, iidx_b)
        issue(gidx_b, rows_b, sem_rb, iidx_b, invr_b, sem_ib)
        drain(gidx_a, rows_a, sem_ra, iidx_a, invr_a, sem_ia, lidx_a)

        @pl.when(s < NSUP - 1)
        def _():
            stage(s + 1)
            cidx(0, gidx_a, lidx_a, iidx_a)
            issue(gidx_a, rows_a, sem_ra, iidx_a, invr_a, sem_ia)

        drain(gidx_b, rows_b, sem_rb, iidx_b, invr_b, sem_ib, lidx_b)
        return carry
    lax.fori_loop(0, NSUP, sup, 0)

    plsc.subcore_barrier()

    pltpu.sync_copy(acc_sh.at[pl.ds(rbase, CHALF // 16)],
                    acc_hbm.at[pl.ds(cid * CHALF + rbase, CHALF // 16)])


def _edge_call(tflat, srcp, dstp, typp, inv):
    return pl.kernel(
        _edge_body,
        out_type=jax.ShapeDtypeStruct((NPAD, D), F32),
        mesh=_mesh,
        compiler_params=_SC_PARAMS,
        scratch_types=[
            pltpu.VMEM((SUP,), I32),
            pltpu.VMEM((SUP,), I32),
            pltpu.VMEM((SUP,), I32),
            pltpu.VMEM((CHUNK,), I32),
            pltpu.VMEM((CHUNK,), I32),
            pltpu.VMEM((CHUNK,), I32),
            pltpu.VMEM((CHUNK,), I32),
            pltpu.VMEM((CHUNK,), I32),
            pltpu.VMEM((CHUNK,), I32),
            pltpu.VMEM((CHUNK, D), F32),
            pltpu.VMEM((CHUNK, D), F32),
            pltpu.VMEM((CHUNK,), F32),
            pltpu.VMEM((CHUNK,), F32),
            pltpu.VMEM((16, D), F32),
            pltpu.SemaphoreType.DMA,
            pltpu.SemaphoreType.DMA,
            pltpu.SemaphoreType.DMA,
            pltpu.SemaphoreType.DMA,
            pltpu.VMEM_SHARED((CHALF + 8, D), F32),
        ],
    )(tflat, srcp, dstp, typp, inv)


def _pool_body(hroot_hbm, acc_hbm, batch_hbm, ps_hbm, pc_hbm,
               bidx_v, h_v, a_v, ones_v, zblk_v, pool_sh, pcnt_sh):
    cid = lax.axis_index("c")
    sid = lax.axis_index("s")
    wid = cid * 16 + sid
    nbase = wid * RPT

    _fill_zero_rows(zblk_v, 112)

    def ob(j, carry):
        for c4 in range(4):
            ones_v[j, pl.ds(16 * c4, 16)] = jnp.ones((16,), F32)
        return carry
    lax.fori_loop(0, 112, ob, 0)

    @pl.when(sid == 0)
    def _():
        pltpu.sync_copy(zblk_v, pool_sh.at[pl.ds(0, 112)])
        pltpu.sync_copy(zblk_v.at[pl.ds(0, GP - 112)], pool_sh.at[pl.ds(112, GP - 112)])
        pltpu.sync_copy(zblk_v, pcnt_sh.at[pl.ds(0, 112)])
        pltpu.sync_copy(zblk_v.at[pl.ds(0, GP - 112)], pcnt_sh.at[pl.ds(112, GP - 112)])

    plsc.subcore_barrier()

    def chunk(c, carry):
        off = nbase + c * 112
        pltpu.sync_copy(batch_hbm.at[pl.ds(off, 112)], bidx_v)
        pltpu.sync_copy(hroot_hbm.at[pl.ds(off, 112)], h_v)
        pltpu.sync_copy(acc_hbm.at[pl.ds(off, 112)], a_v)

        def relub(jr, carry2):
            for c4 in range(4):
                sl = pl.ds(16 * c4, 16)
                h_v[jr, sl] = jnp.maximum(h_v[jr, sl] + a_v[jr, sl], 0.0)
            return carry2
        lax.fori_loop(0, 112, relub, 0)

        pltpu.sync_copy(h_v, pool_sh.at[bidx_v], add=True)
        pltpu.sync_copy(ones_v, pcnt_sh.at[bidx_v], add=True)
        return carry
    lax.fori_loop(0, RPT // 112, chunk, 0)

    plsc.subcore_barrier()

    @pl.when(sid == 0)
    def _():
        pltpu.sync_copy(pool_sh.at[pl.ds(0, 112)], h_v)
        pltpu.sync_copy(h_v, ps_hbm.at[pl.ds(cid * GP, 112)])
        pltpu.sync_copy(pool_sh.at[pl.ds(112, GP - 112)], h_v.at[pl.ds(0, GP - 112)])
        pltpu.sync_copy(h_v.at[pl.ds(0, GP - 112)], ps_hbm.at[pl.ds(cid * GP + 112, GP - 112)])
        pltpu.sync_copy(pcnt_sh.at[pl.ds(0, 112)], h_v)
        pltpu.sync_copy(h_v, pc_hbm.at[pl.ds(cid * GP, 112)])
        pltpu.sync_copy(pcnt_sh.at[pl.ds(112, GP - 112)], h_v.at[pl.ds(0, GP - 112)])
        pltpu.sync_copy(h_v.at[pl.ds(0, GP - 112)], pc_hbm.at[pl.ds(cid * GP + 112, GP - 112)])


def _pool_call(hroot2, acc2, batch_pad):
    return pl.kernel(
        _pool_body,
        out_type=[
            jax.ShapeDtypeStruct((2 * GP, D), F32),
            jax.ShapeDtypeStruct((2 * GP, D), F32),
        ],
        mesh=_mesh,
        compiler_params=_SC_PARAMS,
        scratch_types=[
            pltpu.VMEM((112,), I32),
            pltpu.VMEM((112, D), F32),
            pltpu.VMEM((112, D), F32),
            pltpu.VMEM((112, D), F32),
            pltpu.VMEM((112, D), F32),
            pltpu.VMEM_SHARED((GP, D), F32),
            pltpu.VMEM_SHARED((GP, D), F32),
        ],
    )(hroot2, acc2, batch_pad)


def _fin_body(ps_ref, pc_ref, w_ref, b_ref, out_ref):
    s = ps_ref[0:GP] + ps_ref[GP:2 * GP]
    c = pc_ref[0:GP] + pc_ref[GP:2 * GP]
    g = s[0:G] / jnp.maximum(c[0:G], 1.0)
    out_ref[...] = lax.dot_general(g, w_ref[...], (((1,), (0,)), ((), ())),
                                   preferred_element_type=F32) + b_ref[...]


def _fin_call(ps, pc, lin_W, lin_b):
    return pl.pallas_call(
        _fin_body,
        out_shape=jax.ShapeDtypeStruct((G, lin_W.shape[1]), F32),
    )(ps, pc, lin_W, lin_b.reshape(1, -1))


def kernel(x, edge_index, edge_type, batch, embed_table, W1, root1, b1,
           W2, root2, b2, lin_W, lin_b):
    x_pad = jnp.concatenate([x.astype(I32), jnp.zeros((NPAD - N,), I32)])
    srcp = jnp.concatenate([edge_index[0].astype(I32), jnp.zeros((EPAD - E,), I32)])
    dstp = jnp.concatenate([edge_index[1].astype(I32),
                            jnp.full((EPAD - E,), 4 * NPAD, I32)])
    typp = jnp.concatenate([edge_type.astype(I32), jnp.zeros((EPAD - E,), I32)])
    batch_pad = jnp.concatenate([batch.astype(I32), jnp.full((NPAD - N,), PDUMP, I32)])

    wcat1 = jnp.concatenate([root1, W1[0], W1[1], W1[2]], axis=1)
    wcat2 = jnp.concatenate([root2, W2[0], W2[1], W2[2]], axis=1)

    inv = _cnt_call(srcp, dstp, typp)
    h = _embed_call(x_pad, embed_table)
    hroot1, t1 = _mm_call(h, wcat1, b1.reshape(1, -1))
    acc1 = _edge_call(t1.reshape(3 * NPAD, D), srcp, dstp, typp, inv)
    hroot2, t2 = _nmm_call(hroot1, acc1, wcat2, b2.reshape(1, -1))
    acc2 = _edge_call(t2.reshape(3 * NPAD, D), srcp, dstp, typp, inv)
    ps, pc = _pool_call(hroot2, acc2, batch_pad)
    return _fin_call(ps, pc, lin_W, lin_b)
